# baseline (speedup 1.0000x reference)
import functools

import jax
import jax.numpy as jnp
from jax import lax
from jax.experimental import pallas as pl
from jax.experimental.pallas import tpu as pltpu
from jax.experimental.pallas import tpu_sc as plsc

NC, NS, L = 2, 16, 16
NW = NC * NS


def _make_sc_kernel(b, s, d, ch_rows=16, unroll=8):
    total_rows = b * s
    rpw = total_rows // NW
    n_ch = rpw // ch_rows
    chunk = ch_rows * d
    assert s % rpw == 0
    mesh = plsc.VectorSubcoreMesh(
        core_axis_name="c", subcore_axis_name="s", num_cores=NC, num_subcores=NS
    )

    @functools.partial(
        pl.kernel,
        out_type=jax.ShapeDtypeStruct((total_rows * d,), jnp.float32),
        mesh=mesh,
        scratch_types=(
            [pltpu.VMEM((chunk,), jnp.float32) for _ in range(2)]
            + [pltpu.VMEM((chunk,), jnp.float32) for _ in range(2)]
            + [pltpu.SemaphoreType.DMA for _ in range(6)]
        ),
    )
    def k(x_hbm, pe_hbm, o_hbm, xb0, xb1, pb0, pb1, sx0, sx1, sp0, sp1, so0, so1):
        w = lax.axis_index("s") * NC + lax.axis_index("c")
        bx = w * (rpw * d)
        bp = (w % (s // rpw)) * (rpw * d)
        xb = (xb0, xb1)
        pb = (pb0, pb1)
        sx = (sx0, sx1)
        sp = (sp0, sp1)
        so = (so0, so1)

        def start_in(c):
            bidx = c % 2
            dx = pltpu.async_copy(
                x_hbm.at[pl.ds(bx + c * chunk, chunk)], xb[bidx], sx[bidx]
            )
            dp = pltpu.async_copy(
                pe_hbm.at[pl.ds(bp + c * chunk, chunk)], pb[bidx], sp[bidx]
            )
            return dx, dp

        pend_in = {0: start_in(0)}
        pend_out = {}
        for c in range(n_ch):
            bidx = c % 2
            if c + 1 < n_ch:
                if c >= 1:
                    pend_out.pop(c - 1).wait()
                pend_in[c + 1] = start_in(c + 1)
            dx, dp = pend_in.pop(c)
            dx.wait()
            dp.wait()

            _xb, _pb = xb[bidx], pb[bidx]

            @plsc.parallel_loop(0, chunk, L, unroll=unroll)
            def _(i, _xb=_xb, _pb=_pb):
                _xb[pl.ds(i, L)] = _xb[pl.ds(i, L)] + _pb[pl.ds(i, L)]

            pend_out[c] = pltpu.async_copy(
                xb[bidx], o_hbm.at[pl.ds(bx + c * chunk, chunk)], so[bidx]
            )
        for c in sorted(pend_out):
            pend_out.pop(c).wait()

    return k


@functools.partial(jax.jit, static_argnames=("ch_rows", "unroll"))
def sc_pe_add(x, pe, ch_rows=16, unroll=8):
    b, s, d = x.shape
    k = _make_sc_kernel(b, s, d, ch_rows=ch_rows, unroll=unroll)
    out = k(x.reshape(-1), pe.reshape(-1))
    return out.reshape(b, s, d)


def kernel(x, pe):
    return sc_pe_add(x, pe)


# speedup vs baseline: 1.0068x; 1.0068x over previous
import functools

import jax
import jax.numpy as jnp
from jax import lax
from jax.experimental import pallas as pl
from jax.experimental.pallas import tpu as pltpu
from jax.experimental.pallas import tpu_sc as plsc

NC, NS, L = 2, 16, 16
NW = NC * NS


def _make_sc_kernel(b, s, d, ch_rows=16, unroll=16):
    total_rows = b * s
    rpw = total_rows // NW
    n_ch = rpw // ch_rows
    chunk = ch_rows * d
    assert s % rpw == 0
    mesh = plsc.VectorSubcoreMesh(
        core_axis_name="c", subcore_axis_name="s", num_cores=NC, num_subcores=NS
    )

    @functools.partial(
        pl.kernel,
        out_type=jax.ShapeDtypeStruct((total_rows * d,), jnp.float32),
        mesh=mesh,
        scratch_types=(
            [pltpu.VMEM((chunk,), jnp.float32) for _ in range(6)]
            + [pltpu.SemaphoreType.DMA for _ in range(6)]
        ),
    )
    def k(x_hbm, pe_hbm, o_hbm,
          xb0, xb1, pb0, pb1, ob0, ob1, sx0, sx1, sp0, sp1, so0, so1):
        w = lax.axis_index("s") * NC + lax.axis_index("c")
        bx = w * (rpw * d)
        bp = (w % (s // rpw)) * (rpw * d)
        xb = (xb0, xb1)
        pb = (pb0, pb1)
        ob = (ob0, ob1)
        sx = (sx0, sx1)
        sp = (sp0, sp1)
        so = (so0, so1)

        def start_in(c):
            bidx = c % 2
            dx = pltpu.async_copy(
                x_hbm.at[pl.ds(bx + c * chunk, chunk)], xb[bidx], sx[bidx]
            )
            dp = pltpu.async_copy(
                pe_hbm.at[pl.ds(bp + c * chunk, chunk)], pb[bidx], sp[bidx]
            )
            return dx, dp

        pend_in = {0: start_in(0)}
        pend_out = {}
        for c in range(n_ch):
            bidx = c % 2
            if c + 1 < n_ch:
                pend_in[c + 1] = start_in(c + 1)
            dx, dp = pend_in.pop(c)
            dx.wait()
            dp.wait()
            if c >= 2:
                pend_out.pop(c - 2).wait()

            _xb, _pb, _ob = xb[bidx], pb[bidx], ob[bidx]

            @plsc.parallel_loop(0, chunk, L, unroll=unroll)
            def _(i, _xb=_xb, _pb=_pb, _ob=_ob):
                _ob[pl.ds(i, L)] = _xb[pl.ds(i, L)] + _pb[pl.ds(i, L)]

            pend_out[c] = pltpu.async_copy(
                _ob, o_hbm.at[pl.ds(bx + c * chunk, chunk)], so[bidx]
            )
        for c in sorted(pend_out):
            pend_out.pop(c).wait()

    return k


@functools.partial(jax.jit, static_argnames=("ch_rows", "unroll"))
def sc_pe_add(x, pe, ch_rows=16, unroll=16):
    b, s, d = x.shape
    k = _make_sc_kernel(b, s, d, ch_rows=ch_rows, unroll=unroll)
    out = k(x.reshape(-1), pe.reshape(-1))
    return out.reshape(b, s, d)


def kernel(x, pe):
    return sc_pe_add(x, pe)


# speedup vs baseline: 5.7056x; 5.6673x over previous
import functools

import jax
import jax.numpy as jnp
from jax.experimental import pallas as pl
from jax.experimental.pallas import tpu as pltpu


def _add_block_2d(x_ref, pe_ref, o_ref, *, block_r, seq_len):
    i = pl.program_id(0)
    base = (i * block_r) % seq_len
    o_ref[...] = x_ref[...] + pe_ref[pl.ds(base, block_r), :]


@functools.partial(jax.jit, static_argnames=("block_r",))
def _pe_add(x, pe, block_r=1024):
    b, s, d = x.shape
    x2 = x.reshape(b * s, d)
    out = pl.pallas_call(
        functools.partial(_add_block_2d, block_r=block_r, seq_len=s),
        grid=((b * s) // block_r,),
        in_specs=[
            pl.BlockSpec((block_r, d), lambda i: (i, 0)),
            pl.BlockSpec((s, d), lambda i: (0, 0)),
        ],
        out_specs=pl.BlockSpec((block_r, d), lambda i: (i, 0)),
        out_shape=jax.ShapeDtypeStruct((b * s, d), x.dtype),
    )(x2, pe)
    return out.reshape(b, s, d)


def kernel(x, pe):
    return _pe_add(x, pe, block_r=2048)
